# Initial kernel scaffold; baseline (speedup 1.0000x reference)
import jax
import jax.numpy as jnp
from jax.experimental import pallas as pl


def kernel(x, edge_index, W1, W2):
    raise NotImplementedError("write your pallas kernel here")


# speedup vs baseline: 8.1519x; 8.1519x over previous
import functools

import jax
import jax.numpy as jnp
from jax import lax
from jax.experimental import pallas as pl
from jax.experimental.pallas import tpu as pltpu
from jax.experimental.pallas import tpu_sc as plsc

NC = 2
NS = 16
NW = NC * NS
L = 16

G = 125


def _worker_id():
  return lax.axis_index("s") * NC + lax.axis_index("c")


def _make_deg_kernel(n_nodes, edges_per_worker):
  T = edges_per_worker
  mesh = plsc.VectorSubcoreMesh(core_axis_name="c", subcore_axis_name="s")

  @functools.partial(
      pl.kernel,
      out_type=(
          jax.ShapeDtypeStruct((NW, n_nodes), jnp.float32),
          jax.ShapeDtypeStruct((NW, n_nodes), jnp.float32),
      ),
      mesh=mesh,
      scratch_types=[
          pltpu.VMEM((T,), jnp.int32),
          pltpu.VMEM((n_nodes,), jnp.float32),
      ],
      compiler_params=pltpu.CompilerParams(needs_layout_passes=False),
  )
  def k(src_hbm, dst_hbm, outs_hbm, outd_hbm, idx_v, hist_v):
    wid = _worker_id()
    zeros = jnp.zeros((L,), jnp.float32)
    ones = jnp.ones((L,), jnp.float32)

    def run(idx_hbm, out_hbm):
      def zstep(i, _):
        hist_v[pl.ds(i * L, L)] = zeros
        return 0

      lax.fori_loop(0, n_nodes // L, zstep, 0)
      pltpu.sync_copy(idx_hbm.at[wid], idx_v)

      def astep(i, _):
        iv = idx_v[pl.ds(i * L, L)]
        plsc.addupdate_scatter(hist_v, [iv], ones)
        return 0

      lax.fori_loop(0, T // L, astep, 0)
      pltpu.sync_copy(hist_v, out_hbm.at[wid])

    run(src_hbm, outs_hbm)
    run(dst_hbm, outd_hbm)

  return k


def _make_gs_kernel(n_nodes, n_chunks, d):
  rpt = (n_nodes // NS) // 8 * 8
  rem_start = rpt * NS
  rem = n_nodes - rem_start
  assert rem % 8 == 0
  mesh = plsc.VectorSubcoreMesh(core_axis_name="c", subcore_axis_name="s")

  @functools.partial(
      pl.kernel,
      out_type=jax.ShapeDtypeStruct((NC, n_nodes, d), jnp.float32),
      mesh=mesh,
      scratch_types=[
          pltpu.VMEM((n_chunks, G), jnp.int32),
          pltpu.VMEM((n_chunks, G), jnp.int32),
          pltpu.VMEM((2, G, d), jnp.float32),
          pltpu.VMEM_SHARED((n_nodes, d), jnp.float32),
          pltpu.SemaphoreType.DMA,
          pltpu.SemaphoreType.DMA,
      ],
      compiler_params=pltpu.CompilerParams(use_tc_tiling_on_sc=False),
  )
  def k(h_hbm, sidx_hbm, didx_hbm, zero_hbm, out_hbm, sidx_v, didx_v, buf_v,
        acc_sh, sem0, sem1):
    cid = lax.axis_index("c")
    sid = lax.axis_index("s")
    wid = _worker_id()

    pltpu.sync_copy(sidx_hbm.at[wid], sidx_v)
    pltpu.sync_copy(didx_hbm.at[wid], didx_v)

    base = pl.multiple_of(sid * rpt, 8)
    pltpu.sync_copy(zero_hbm.at[pl.ds(base, rpt)], acc_sh.at[pl.ds(base, rpt)])
    if rem:
      @pl.when(sid == NS - 1)
      def _():
        pltpu.sync_copy(zero_hbm.at[pl.ds(rem_start, rem)],
                        acc_sh.at[pl.ds(rem_start, rem)])
    plsc.subcore_barrier()

    def chunk(j, _):
      pltpu.async_copy(h_hbm.at[sidx_v.at[j]], buf_v.at[0], sem0).wait()
      pltpu.sync_copy(buf_v.at[0], acc_sh.at[didx_v.at[j]], add=True)
      return 0

    lax.fori_loop(0, n_chunks, chunk, 0)
    plsc.subcore_barrier()

    pltpu.sync_copy(acc_sh.at[pl.ds(base, rpt)],
                    out_hbm.at[cid, pl.ds(base, rpt)])
    if rem:
      @pl.when(sid == NS - 1)
      def _():
        pltpu.sync_copy(acc_sh.at[pl.ds(rem_start, rem)],
                        out_hbm.at[cid, pl.ds(rem_start, rem)])

  return k


def _norm_body(ds_ref, dd_ref, ns_ref, nd_ref):
  ns_ref[...] = lax.rsqrt(
      jnp.maximum(jnp.sum(ds_ref[...], axis=0), 1.0))[:, None]
  nd_ref[...] = lax.rsqrt(
      jnp.maximum(jnp.sum(dd_ref[...], axis=0), 1.0))[:, None]


def _norms(degs, degd):
  n = degs.shape[1]
  return pl.pallas_call(
      _norm_body,
      out_shape=[
          jax.ShapeDtypeStruct((n, 1), jnp.float32),
          jax.ShapeDtypeStruct((n, 1), jnp.float32),
      ],
  )(degs, degd)


def _mm1_body(x_ref, w_ref, ns_ref, ha_ref, hb_ref):
  h = w_ref.shape[1]
  xs = x_ref[...] * ns_ref[...]
  full = jnp.dot(xs, w_ref[...], preferred_element_type=jnp.float32)
  ha_ref[...] = full[:, : h // 2]
  hb_ref[...] = full[:, h // 2 :]


def _mm2_body(pa_ref, pb_ref, ns_ref, nd_ref, w_ref, o_ref):
  h = w_ref.shape[0]
  nd = nd_ref[...]
  ns = ns_ref[...]
  h1a = jnp.maximum((pa_ref[0] + pa_ref[1]) * nd, 0.0) * ns
  h1b = jnp.maximum((pb_ref[0] + pb_ref[1]) * nd, 0.0) * ns
  o_ref[...] = (
      jnp.dot(h1a, w_ref[: h // 2], preferred_element_type=jnp.float32)
      + jnp.dot(h1b, w_ref[h // 2 :], preferred_element_type=jnp.float32))


def _fin_body(p_ref, nd_ref, o_ref):
  o_ref[...] = (p_ref[0] + p_ref[1]) * nd_ref[...]


def _mm1(x, w1, ns, blk):
  n, f = x.shape
  h = w1.shape[1]
  return pl.pallas_call(
      _mm1_body,
      grid=(n // blk,),
      in_specs=[
          pl.BlockSpec((blk, f), lambda i: (i, 0)),
          pl.BlockSpec((f, h), lambda i: (0, 0)),
          pl.BlockSpec((blk, 1), lambda i: (i, 0)),
      ],
      out_specs=[
          pl.BlockSpec((blk, h // 2), lambda i: (i, 0)),
          pl.BlockSpec((blk, h // 2), lambda i: (i, 0)),
      ],
      out_shape=[
          jax.ShapeDtypeStruct((n, h // 2), jnp.float32),
          jax.ShapeDtypeStruct((n, h // 2), jnp.float32),
      ],
  )(x, w1, ns)


def _mm2(pa, pb, ns, nd, w2, blk):
  n = pa.shape[1]
  hh = pa.shape[2]
  o = w2.shape[1]
  return pl.pallas_call(
      _mm2_body,
      grid=(n // blk,),
      in_specs=[
          pl.BlockSpec((NC, blk, hh), lambda i: (0, i, 0)),
          pl.BlockSpec((NC, blk, hh), lambda i: (0, i, 0)),
          pl.BlockSpec((blk, 1), lambda i: (i, 0)),
          pl.BlockSpec((blk, 1), lambda i: (i, 0)),
          pl.BlockSpec((2 * hh, o), lambda i: (0, 0)),
      ],
      out_specs=pl.BlockSpec((blk, o), lambda i: (i, 0)),
      out_shape=jax.ShapeDtypeStruct((n, o), jnp.float32),
  )(pa, pb, ns, nd, w2)


def _fin(p, nd, blk):
  n = p.shape[1]
  o = p.shape[2]
  return pl.pallas_call(
      _fin_body,
      grid=(n // blk,),
      in_specs=[
          pl.BlockSpec((NC, blk, o), lambda i: (0, i, 0)),
          pl.BlockSpec((blk, 1), lambda i: (i, 0)),
      ],
      out_specs=pl.BlockSpec((blk, o), lambda i: (i, 0)),
      out_shape=jax.ShapeDtypeStruct((n, o), jnp.float32),
  )(p, nd)


@jax.jit
def kernel(x, edge_index, W1, W2):
  n, f = x.shape
  h = W1.shape[1]
  o = W2.shape[1]
  e = edge_index.shape[1]

  assert e % (NW * G) == 0 and n % NS == 0 and n % L == 0
  assert (e // NW) % L == 0
  t = e // NW
  c = t // G

  ei = edge_index.astype(jnp.int32)
  src_t = ei[0].reshape(NW, t)
  dst_t = ei[1].reshape(NW, t)
  src_c = ei[0].reshape(NW, c, G)
  dst_c = ei[1].reshape(NW, c, G)

  degs, degd = _make_deg_kernel(n, t)(src_t, dst_t)
  ns, nd = _norms(degs, degd)
  ha, hb = _mm1(x, W1, ns, 2000)

  zero_h = jnp.zeros((n, h // 2), jnp.float32)
  gs = _make_gs_kernel(n, c, h // 2)
  p1a = gs(ha, src_c, dst_c, zero_h)
  p1b = gs(hb, src_c, dst_c, zero_h)
  h2 = _mm2(p1a, p1b, ns, nd, W2, 2000)

  zero_o = jnp.zeros((n, o), jnp.float32)
  p2 = _make_gs_kernel(n, c, o)(h2, src_c, dst_c, zero_o)
  return _fin(p2, nd, 2000)


# speedup vs baseline: 9.7752x; 1.1991x over previous
import functools

import jax
import jax.numpy as jnp
from jax import lax
from jax.experimental import pallas as pl
from jax.experimental.pallas import tpu as pltpu
from jax.experimental.pallas import tpu_sc as plsc

NC = 2
NS = 16
NW = NC * NS
L = 16

G = 125


def _worker_id():
  return lax.axis_index("s") * NC + lax.axis_index("c")


def _make_deg_kernel(n_nodes, edges_per_worker):
  T = edges_per_worker
  mesh = plsc.VectorSubcoreMesh(core_axis_name="c", subcore_axis_name="s")

  @functools.partial(
      pl.kernel,
      out_type=(
          jax.ShapeDtypeStruct((NW, n_nodes), jnp.float32),
          jax.ShapeDtypeStruct((NW, n_nodes), jnp.float32),
      ),
      mesh=mesh,
      scratch_types=[
          pltpu.VMEM((T,), jnp.int32),
          pltpu.VMEM((n_nodes,), jnp.float32),
      ],
      compiler_params=pltpu.CompilerParams(needs_layout_passes=False),
  )
  def k(src_hbm, dst_hbm, outs_hbm, outd_hbm, idx_v, hist_v):
    wid = _worker_id()
    zeros = jnp.zeros((L,), jnp.float32)
    ones = jnp.ones((L,), jnp.float32)

    def run(idx_hbm, out_hbm):
      def zstep(i, _):
        hist_v[pl.ds(i * L, L)] = zeros
        return 0

      lax.fori_loop(0, n_nodes // L, zstep, 0)
      pltpu.sync_copy(idx_hbm.at[wid], idx_v)

      def astep(i, _):
        iv = idx_v[pl.ds(i * L, L)]
        plsc.addupdate_scatter(hist_v, [iv], ones)
        return 0

      lax.fori_loop(0, T // L, astep, 0)
      pltpu.sync_copy(hist_v, out_hbm.at[wid])

    run(src_hbm, outs_hbm)
    run(dst_hbm, outd_hbm)

  return k


def _make_gs_kernel(n_nodes, n_chunks, d):
  rpt = (n_nodes // NS) // 8 * 8
  rem_start = rpt * NS
  rem = n_nodes - rem_start
  assert rem % 8 == 0
  mesh = plsc.VectorSubcoreMesh(core_axis_name="c", subcore_axis_name="s")

  @functools.partial(
      pl.kernel,
      out_type=jax.ShapeDtypeStruct((NC, n_nodes, d), jnp.float32),
      mesh=mesh,
      scratch_types=[
          pltpu.VMEM((n_chunks, G), jnp.int32),
          pltpu.VMEM((n_chunks, G), jnp.int32),
          pltpu.VMEM((2, G, d), jnp.float32),
          pltpu.VMEM_SHARED((n_nodes, d), jnp.float32),
          pltpu.SemaphoreType.DMA,
          pltpu.SemaphoreType.DMA,
          pltpu.SemaphoreType.DMA,
          pltpu.SemaphoreType.DMA,
      ],
      compiler_params=pltpu.CompilerParams(use_tc_tiling_on_sc=False),
  )
  def k(h_hbm, sidx_hbm, didx_hbm, zero_hbm, out_hbm, sidx_v, didx_v, buf_v,
        acc_sh, gsem0, gsem1, ssem0, ssem1):
    cid = lax.axis_index("c")
    sid = lax.axis_index("s")
    wid = _worker_id()

    pltpu.sync_copy(sidx_hbm.at[wid], sidx_v)
    pltpu.sync_copy(didx_hbm.at[wid], didx_v)

    base = pl.multiple_of(sid * rpt, 8)
    pltpu.sync_copy(zero_hbm.at[pl.ds(base, rpt)], acc_sh.at[pl.ds(base, rpt)])
    if rem:
      @pl.when(sid == NS - 1)
      def _():
        pltpu.sync_copy(zero_hbm.at[pl.ds(rem_start, rem)],
                        acc_sh.at[pl.ds(rem_start, rem)])
    plsc.subcore_barrier()

    gsems = [gsem0, gsem1]
    ssems = [ssem0, ssem1]
    assert n_chunks % 2 == 0
    pltpu.async_copy(h_hbm.at[sidx_v.at[0]], buf_v.at[0], gsem0)

    def body(g, _):
      for b in range(2):
        j = g * 2 + b
        pltpu.make_async_copy(h_hbm.at[sidx_v.at[j]], buf_v.at[b],
                              gsems[b]).wait()

        @pl.when(j >= 1)
        def _():
          pltpu.make_async_copy(buf_v.at[1 - b],
                                acc_sh.at[didx_v.at[j - 1]],
                                ssems[1 - b]).wait()

        @pl.when(j + 1 < n_chunks)
        def _():
          pltpu.async_copy(h_hbm.at[sidx_v.at[j + 1]], buf_v.at[1 - b],
                           gsems[1 - b])

        pltpu.async_copy(buf_v.at[b], acc_sh.at[didx_v.at[j]], ssems[b],
                         add=True)
      return 0

    lax.fori_loop(0, n_chunks // 2, body, 0)
    last = n_chunks - 1
    pltpu.make_async_copy(buf_v.at[last % 2], acc_sh.at[didx_v.at[last]],
                          ssems[last % 2]).wait()
    plsc.subcore_barrier()

    pltpu.sync_copy(acc_sh.at[pl.ds(base, rpt)],
                    out_hbm.at[cid, pl.ds(base, rpt)])
    if rem:
      @pl.when(sid == NS - 1)
      def _():
        pltpu.sync_copy(acc_sh.at[pl.ds(rem_start, rem)],
                        out_hbm.at[cid, pl.ds(rem_start, rem)])

  return k


def _norm_body(ds_ref, dd_ref, ns_ref, nd_ref):
  ns_ref[...] = lax.rsqrt(
      jnp.maximum(jnp.sum(ds_ref[...], axis=0), 1.0))[:, None]
  nd_ref[...] = lax.rsqrt(
      jnp.maximum(jnp.sum(dd_ref[...], axis=0), 1.0))[:, None]


def _norms(degs, degd):
  n = degs.shape[1]
  return pl.pallas_call(
      _norm_body,
      out_shape=[
          jax.ShapeDtypeStruct((n, 1), jnp.float32),
          jax.ShapeDtypeStruct((n, 1), jnp.float32),
      ],
  )(degs, degd)


def _mm1_body(x_ref, w_ref, ns_ref, ha_ref, hb_ref):
  h = w_ref.shape[1]
  xs = x_ref[...] * ns_ref[...]
  full = jnp.dot(xs, w_ref[...], preferred_element_type=jnp.float32)
  ha_ref[...] = full[:, : h // 2]
  hb_ref[...] = full[:, h // 2 :]


def _mm2_body(pa_ref, pb_ref, ns_ref, nd_ref, w_ref, o_ref):
  h = w_ref.shape[0]
  nd = nd_ref[...]
  ns = ns_ref[...]
  h1a = jnp.maximum((pa_ref[0] + pa_ref[1]) * nd, 0.0) * ns
  h1b = jnp.maximum((pb_ref[0] + pb_ref[1]) * nd, 0.0) * ns
  o_ref[...] = (
      jnp.dot(h1a, w_ref[: h // 2], preferred_element_type=jnp.float32)
      + jnp.dot(h1b, w_ref[h // 2 :], preferred_element_type=jnp.float32))


def _fin_body(p_ref, nd_ref, o_ref):
  o_ref[...] = (p_ref[0] + p_ref[1]) * nd_ref[...]


def _mm1(x, w1, ns, blk):
  n, f = x.shape
  h = w1.shape[1]
  return pl.pallas_call(
      _mm1_body,
      grid=(n // blk,),
      in_specs=[
          pl.BlockSpec((blk, f), lambda i: (i, 0)),
          pl.BlockSpec((f, h), lambda i: (0, 0)),
          pl.BlockSpec((blk, 1), lambda i: (i, 0)),
      ],
      out_specs=[
          pl.BlockSpec((blk, h // 2), lambda i: (i, 0)),
          pl.BlockSpec((blk, h // 2), lambda i: (i, 0)),
      ],
      out_shape=[
          jax.ShapeDtypeStruct((n, h // 2), jnp.float32),
          jax.ShapeDtypeStruct((n, h // 2), jnp.float32),
      ],
  )(x, w1, ns)


def _mm2(pa, pb, ns, nd, w2, blk):
  n = pa.shape[1]
  hh = pa.shape[2]
  o = w2.shape[1]
  return pl.pallas_call(
      _mm2_body,
      grid=(n // blk,),
      in_specs=[
          pl.BlockSpec((NC, blk, hh), lambda i: (0, i, 0)),
          pl.BlockSpec((NC, blk, hh), lambda i: (0, i, 0)),
          pl.BlockSpec((blk, 1), lambda i: (i, 0)),
          pl.BlockSpec((blk, 1), lambda i: (i, 0)),
          pl.BlockSpec((2 * hh, o), lambda i: (0, 0)),
      ],
      out_specs=pl.BlockSpec((blk, o), lambda i: (i, 0)),
      out_shape=jax.ShapeDtypeStruct((n, o), jnp.float32),
  )(pa, pb, ns, nd, w2)


def _fin(p, nd, blk):
  n = p.shape[1]
  o = p.shape[2]
  return pl.pallas_call(
      _fin_body,
      grid=(n // blk,),
      in_specs=[
          pl.BlockSpec((NC, blk, o), lambda i: (0, i, 0)),
          pl.BlockSpec((blk, 1), lambda i: (i, 0)),
      ],
      out_specs=pl.BlockSpec((blk, o), lambda i: (i, 0)),
      out_shape=jax.ShapeDtypeStruct((n, o), jnp.float32),
  )(p, nd)


@jax.jit
def kernel(x, edge_index, W1, W2):
  n, f = x.shape
  h = W1.shape[1]
  o = W2.shape[1]
  e = edge_index.shape[1]

  assert e % (NW * G) == 0 and n % NS == 0 and n % L == 0
  assert (e // NW) % L == 0
  t = e // NW
  c = t // G

  ei = edge_index.astype(jnp.int32)
  src_t = ei[0].reshape(NW, t)
  dst_t = ei[1].reshape(NW, t)
  src_c = ei[0].reshape(NW, c, G)
  dst_c = ei[1].reshape(NW, c, G)

  degs, degd = _make_deg_kernel(n, t)(src_t, dst_t)
  ns, nd = _norms(degs, degd)
  ha, hb = _mm1(x, W1, ns, 2000)

  zero_h = jnp.zeros((n, h // 2), jnp.float32)
  gs = _make_gs_kernel(n, c, h // 2)
  p1a = gs(ha, src_c, dst_c, zero_h)
  p1b = gs(hb, src_c, dst_c, zero_h)
  h2 = _mm2(p1a, p1b, ns, nd, W2, 2000)

  zero_o = jnp.zeros((n, o), jnp.float32)
  p2 = _make_gs_kernel(n, c, o)(h2, src_c, dst_c, zero_o)
  return _fin(p2, nd, 2000)
